# baseline (speedup 1.0000x reference)
import functools

import jax
import jax.numpy as jnp
from jax import lax
from jax.experimental import pallas as pl
from jax.experimental.pallas import tpu as pltpu
from jax.experimental.pallas import tpu_sc as plsc

_V = 100000
_D = 128
_B = 16384
_XMAX = 100.0
_ALPHA = 0.75

_NC = 2
_NS = 16
_L = 16
_NW = _NC * _NS
_BPW = _B // _NW
_CH = 128
_NCH = _BPW // _CH
_TP = 17

_LN2 = 0.6931471805599453
_LN_XMAX = 4.605170185988092
_SQRT2 = 1.4142135623730951


_GDN = lax.GatherDimensionNumbers(
    offset_dims=(), collapsed_slice_dims=(0,), start_index_map=(0,))


def _permute(v, idx):
    return lax.gather(v, idx[:, None], _GDN, (1,),
                      mode=lax.GatherScatterMode.PROMISE_IN_BOUNDS)


def _ln16(v):
    bits = plsc.bitcast(v, jnp.int32)
    e = ((bits >> 23) & 0xFF) - 127
    m = plsc.bitcast((bits & 0x7FFFFF) | 0x3F800000, jnp.float32)
    big = m > _SQRT2
    m = jnp.where(big, m * 0.5, m)
    ef = (e + jnp.where(big, 1, 0)).astype(jnp.float32)
    s = (m - 1.0) / (m + 1.0)
    s2 = s * s
    p = 1.0 + s2 * ((1.0 / 3.0) + s2 * ((1.0 / 5.0) + s2 * (1.0 / 7.0)))
    return ef * _LN2 + 2.0 * s * p


@functools.partial(
    pl.kernel,
    out_type=jax.ShapeDtypeStruct((_NW, _L), jnp.float32),
    mesh=plsc.VectorSubcoreMesh(core_axis_name="c", subcore_axis_name="s"),
    compiler_params=pltpu.CompilerParams(needs_layout_passes=False),
    scratch_types=[
        pltpu.VMEM((_BPW,), jnp.int32),
        pltpu.VMEM((_BPW,), jnp.int32),
        pltpu.VMEM((_BPW,), jnp.float32),
        pltpu.VMEM((2, _CH, _D), jnp.float32),
        pltpu.VMEM((2, _CH, _D), jnp.float32),
        pltpu.VMEM((2, _CH), jnp.float32),
        pltpu.VMEM((2, _CH), jnp.float32),
        pltpu.VMEM((_L,), jnp.float32),
        pltpu.VMEM((_L * _TP,), jnp.float32),
        pltpu.VMEM((_L * _TP,), jnp.float32),
        pltpu.SemaphoreType.DMA,
        pltpu.SemaphoreType.DMA,
    ],
)
def _glove_sc(i_hbm, j_hbm, x_hbm, w_hbm, wt_hbm, b_hbm, bt_hbm, out_hbm,
              iv, jv, xv, wib, wjb, bib, bjb, accv, tbuf, tbuf2, sem0, sem1):
    wid = lax.axis_index("s") * _NC + lax.axis_index("c")
    base = wid * _BPW
    pltpu.sync_copy(i_hbm.at[pl.ds(base, _BPW)], iv)
    pltpu.sync_copy(j_hbm.at[pl.ds(base, _BPW)], jv)
    pltpu.sync_copy(x_hbm.at[pl.ds(base, _BPW)], xv)

    sems = (sem0, sem1)

    def issue(c):
        slot = c % 2
        idx_i = iv.at[pl.ds(c * _CH, _CH)]
        idx_j = jv.at[pl.ds(c * _CH, _CH)]
        sem = sems[slot]
        return (
            pltpu.async_copy(w_hbm.at[idx_i], wib.at[slot], sem),
            pltpu.async_copy(wt_hbm.at[idx_j], wjb.at[slot], sem),
            pltpu.async_copy(b_hbm.at[idx_i], bib.at[slot], sem),
            pltpu.async_copy(bt_hbm.at[idx_j], bjb.at[slot], sem),
        )

    acc = jnp.zeros((_L,), jnp.float32)
    pending = {0: issue(0)}
    for c in range(_NCH):
        slot = c % 2
        if c + 1 < _NCH:
            pending[c + 1] = issue(c + 1)
        for cp in pending.pop(c):
            cp.wait()

        wiv = wib.at[slot]
        wjv = wjb.at[slot]

        def gbody(t, acc, slot=slot, wiv=wiv, wjv=wjv):
            lane = lax.iota(jnp.int32, _L)
            colbase = lane * _TP

            def pair_sum(p):
                m = [wiv[p, pl.ds(k * _L, _L)] * wjv[p, pl.ds(k * _L, _L)]
                     for k in range(_D // _L)]
                m = [m[2 * t] + m[2 * t + 1] for t in range(4)]
                m = [m[2 * t] + m[2 * t + 1] for t in range(2)]
                return m[0] + m[1]

            def group(g, buf):
                for l in range(_L):
                    plsc.store_scatter(buf, [colbase + l],
                                       pair_sum(g * _L + l))
                terms = [plsc.load_gather(buf, [lane + k * _TP])
                         for k in range(_L)]
                while len(terms) > 1:
                    terms = [terms[2 * t] + terms[2 * t + 1]
                             for t in range(len(terms) // 2)]
                dots = terms[0]
                sl = pl.ds(g * _L, _L)
                xsl = pl.ds(c * _CH + g * _L, _L)
                lnx = _ln16(xv[xsl])
                f = jnp.minimum(jnp.exp(_ALPHA * (lnx - _LN_XMAX)), 1.0)
                diff = dots + bib[slot, sl] + bjb[slot, sl] - lnx
                return f * diff * diff

            return acc + group(2 * t, tbuf) + group(2 * t + 1, tbuf2)

        acc = lax.fori_loop(0, _CH // _L // 2, gbody, acc)
    accv[...] = acc
    pltpu.sync_copy(accv, out_hbm.at[wid])


def kernel(i, j, x, weight, weight_tilde, bias, bias_tilde):
    parts = _glove_sc(i, j, x, weight, weight_tilde, bias, bias_tilde)
    return jnp.sum(parts) / _B


# speedup vs baseline: 1.1188x; 1.1188x over previous
import functools

import jax
import jax.numpy as jnp
from jax import lax
from jax.experimental import pallas as pl
from jax.experimental.pallas import tpu as pltpu
from jax.experimental.pallas import tpu_sc as plsc

_V = 100000
_D = 128
_B = 16384
_XMAX = 100.0
_ALPHA = 0.75

_NC = 2
_NS = 16
_L = 16
_NW = _NC * _NS
_BPW = _B // _NW
_CH = 128
_NCH = _BPW // _CH
_TP = 17

_LN2 = 0.6931471805599453
_LN_XMAX = 4.605170185988092
_SQRT2 = 1.4142135623730951


_GDN = lax.GatherDimensionNumbers(
    offset_dims=(), collapsed_slice_dims=(0,), start_index_map=(0,))


def _permute(v, idx):
    return lax.gather(v, idx[:, None], _GDN, (1,),
                      mode=lax.GatherScatterMode.PROMISE_IN_BOUNDS)


def _ln16(v):
    bits = plsc.bitcast(v, jnp.int32)
    e = ((bits >> 23) & 0xFF) - 127
    m = plsc.bitcast((bits & 0x7FFFFF) | 0x3F800000, jnp.float32)
    big = m > _SQRT2
    m = jnp.where(big, m * 0.5, m)
    ef = (e + jnp.where(big, 1, 0)).astype(jnp.float32)
    s = (m - 1.0) / (m + 1.0)
    s2 = s * s
    p = 1.0 + s2 * ((1.0 / 3.0) + s2 * ((1.0 / 5.0) + s2 * (1.0 / 7.0)))
    return ef * _LN2 + 2.0 * s * p


@functools.partial(
    pl.kernel,
    out_type=jax.ShapeDtypeStruct((_NW, _L), jnp.float32),
    mesh=plsc.VectorSubcoreMesh(core_axis_name="c", subcore_axis_name="s"),
    compiler_params=pltpu.CompilerParams(needs_layout_passes=False),
    scratch_types=[
        pltpu.VMEM((_BPW,), jnp.int32),
        pltpu.VMEM((_BPW,), jnp.int32),
        pltpu.VMEM((_BPW,), jnp.float32),
        pltpu.VMEM((2, _CH, _D), jnp.float32),
        pltpu.VMEM((2, _CH, _D), jnp.float32),
        pltpu.VMEM((2, _CH), jnp.float32),
        pltpu.VMEM((2, _CH), jnp.float32),
        pltpu.VMEM((_L,), jnp.float32),
        pltpu.VMEM((_L * _TP,), jnp.float32),
        pltpu.VMEM((_L * _TP,), jnp.float32),
        pltpu.SemaphoreType.DMA,
        pltpu.SemaphoreType.DMA,
    ],
)
def _glove_sc(i_hbm, j_hbm, x_hbm, w_hbm, wt_hbm, b_hbm, bt_hbm, out_hbm,
              iv, jv, xv, wib, wjb, bib, bjb, accv, tbuf, tbuf2, sem0, sem1):
    wid = lax.axis_index("s") * _NC + lax.axis_index("c")
    base = wid * _BPW
    with jax.named_scope("stage_idx"):
        c0 = pltpu.async_copy(i_hbm.at[pl.ds(base, _BPW)], iv, sem0)
        c1 = pltpu.async_copy(j_hbm.at[pl.ds(base, _BPW)], jv, sem0)
        c2 = pltpu.async_copy(x_hbm.at[pl.ds(base, _BPW)], xv, sem0)
        c0.wait()
        c1.wait()
        c2.wait()

    sems = (sem0, sem1)

    def issue(c):
        slot = c % 2
        idx_i = iv.at[pl.ds(c * _CH, _CH)]
        idx_j = jv.at[pl.ds(c * _CH, _CH)]
        sem = sems[slot]
        return (
            pltpu.async_copy(w_hbm.at[idx_i], wib.at[slot], sem),
            pltpu.async_copy(wt_hbm.at[idx_j], wjb.at[slot], sem),
            pltpu.async_copy(b_hbm.at[idx_i], bib.at[slot], sem),
            pltpu.async_copy(bt_hbm.at[idx_j], bjb.at[slot], sem),
        )

    acc = jnp.zeros((_L,), jnp.float32)
    pending = {0: issue(0)}
    for c in range(_NCH):
        slot = c % 2
        if c + 1 < _NCH:
            pending[c + 1] = issue(c + 1)
        with jax.named_scope(f"wait{c}"):
            for cp in pending.pop(c):
                cp.wait()

        wiv = wib.at[slot]
        wjv = wjb.at[slot]

        def gbody(g, acc, slot=slot, wiv=wiv, wjv=wjv):
            lane = lax.iota(jnp.int32, _L)
            colbase = lane * _TP

            def pair_sum(p):
                m = [wiv[p, pl.ds(k * _L, _L)] * wjv[p, pl.ds(k * _L, _L)]
                     for k in range(_D // _L)]
                m = [m[2 * t] + m[2 * t + 1] for t in range(4)]
                m = [m[2 * t] + m[2 * t + 1] for t in range(2)]
                return m[0] + m[1]

            for l in range(_L):
                plsc.store_scatter(tbuf, [colbase + l], pair_sum(g * _L + l))
            terms = [plsc.load_gather(tbuf, [lane + k * _TP])
                     for k in range(_L)]
            while len(terms) > 1:
                terms = [terms[2 * t] + terms[2 * t + 1]
                         for t in range(len(terms) // 2)]
            dots = terms[0]
            sl = pl.ds(g * _L, _L)
            xsl = pl.ds(c * _CH + g * _L, _L)
            lnx = _ln16(xv[xsl])
            f = jnp.minimum(jnp.exp(_ALPHA * (lnx - _LN_XMAX)), 1.0)
            diff = dots + bib[slot, sl] + bjb[slot, sl] - lnx
            return acc + f * diff * diff

        with jax.named_scope(f"compute{c}"):
            acc = lax.fori_loop(0, _CH // _L, gbody, acc)
    accv[...] = acc
    pltpu.sync_copy(accv, out_hbm.at[wid])


def kernel(i, j, x, weight, weight_tilde, bias, bias_tilde):
    parts = _glove_sc(i, j, x, weight, weight_tilde, bias, bias_tilde)
    return jnp.sum(parts) / _B


# speedup vs baseline: 1.2391x; 1.1075x over previous
import functools

import jax
import jax.numpy as jnp
from jax import lax
from jax.experimental import pallas as pl
from jax.experimental.pallas import tpu as pltpu
from jax.experimental.pallas import tpu_sc as plsc

_V = 100000
_D = 128
_B = 16384
_XMAX = 100.0
_ALPHA = 0.75

_NC = 2
_NS = 16
_L = 16
_NW = _NC * _NS
_BPW = _B // _NW
_CH = 128
_NCH = _BPW // _CH
_TP = 17

_LN2 = 0.6931471805599453
_LN_XMAX = 4.605170185988092
_SQRT2 = 1.4142135623730951


_GDN = lax.GatherDimensionNumbers(
    offset_dims=(), collapsed_slice_dims=(0,), start_index_map=(0,))


def _permute(v, idx):
    return lax.gather(v, idx[:, None], _GDN, (1,),
                      mode=lax.GatherScatterMode.PROMISE_IN_BOUNDS)


def _ln16(v):
    bits = plsc.bitcast(v, jnp.int32)
    e = ((bits >> 23) & 0xFF) - 127
    m = plsc.bitcast((bits & 0x7FFFFF) | 0x3F800000, jnp.float32)
    big = m > _SQRT2
    m = jnp.where(big, m * 0.5, m)
    ef = (e + jnp.where(big, 1, 0)).astype(jnp.float32)
    s = (m - 1.0) / (m + 1.0)
    s2 = s * s
    p = 1.0 + s2 * ((1.0 / 3.0) + s2 * ((1.0 / 5.0) + s2 * (1.0 / 7.0)))
    return ef * _LN2 + 2.0 * s * p


@functools.partial(
    pl.kernel,
    out_type=jax.ShapeDtypeStruct((_NW, _L), jnp.float32),
    mesh=plsc.VectorSubcoreMesh(core_axis_name="c", subcore_axis_name="s"),
    compiler_params=pltpu.CompilerParams(needs_layout_passes=False),
    scratch_types=[
        pltpu.VMEM((_BPW,), jnp.int32),
        pltpu.VMEM((_BPW,), jnp.int32),
        pltpu.VMEM((_BPW,), jnp.float32),
        pltpu.VMEM((2, _CH, _D), jnp.float32),
        pltpu.VMEM((2, _CH, _D), jnp.float32),
        pltpu.VMEM((2, _CH), jnp.float32),
        pltpu.VMEM((2, _CH), jnp.float32),
        pltpu.VMEM((_L,), jnp.float32),
        pltpu.VMEM((_L * _TP,), jnp.float32),
        pltpu.VMEM((_L * _TP,), jnp.float32),
        pltpu.SemaphoreType.DMA,
        pltpu.SemaphoreType.DMA,
    ],
)
def _glove_sc(i_hbm, j_hbm, x_hbm, w_hbm, wt_hbm, b_hbm, bt_hbm, out_hbm,
              iv, jv, xv, wib, wjb, bib, bjb, accv, tbuf, tbuf2, sem0, sem1):
    wid = lax.axis_index("s") * _NC + lax.axis_index("c")
    base = wid * _BPW
    with jax.named_scope("stage_idx"):
        c0 = pltpu.async_copy(i_hbm.at[pl.ds(base, _BPW)], iv, sem0)
        c1 = pltpu.async_copy(j_hbm.at[pl.ds(base, _BPW)], jv, sem0)
        c2 = pltpu.async_copy(x_hbm.at[pl.ds(base, _BPW)], xv, sem0)
        c0.wait()
        c1.wait()
        c2.wait()

    sems = (sem0, sem1)

    def issue(c):
        slot = c % 2
        idx_i = iv.at[pl.ds(c * _CH, _CH)]
        idx_j = jv.at[pl.ds(c * _CH, _CH)]
        sem = sems[slot]
        return (
            pltpu.async_copy(w_hbm.at[idx_i], wib.at[slot], sem),
            pltpu.async_copy(wt_hbm.at[idx_j], wjb.at[slot], sem),
            pltpu.async_copy(b_hbm.at[idx_i], bib.at[slot], sem),
            pltpu.async_copy(bt_hbm.at[idx_j], bjb.at[slot], sem),
        )

    acc = jnp.zeros((_L,), jnp.float32)
    pending = {0: issue(0)}
    for c in range(_NCH):
        slot = c % 2
        if c + 1 < _NCH:
            pending[c + 1] = issue(c + 1)
        with jax.named_scope(f"wait{c}"):
            for cp in pending.pop(c):
                cp.wait()

        wiv = wib.at[slot]
        wjv = wjb.at[slot]

        def gbody(g, acc, slot=slot, wiv=wiv, wjv=wjv):
            lane = lax.iota(jnp.int32, _L)
            colbase = lane * _TP

            def pair_loads(p):
                return ([wiv[p, pl.ds(k * _L, _L)] for k in range(_D // _L)],
                        [wjv[p, pl.ds(k * _L, _L)] for k in range(_D // _L)])

            def pair_alu(wi, wj):
                m = [a * b for a, b in zip(wi, wj)]
                m = [m[2 * t] + m[2 * t + 1] for t in range(4)]
                m = [m[2 * t] + m[2 * t + 1] for t in range(2)]
                return m[0] + m[1]

            held = pair_loads(g * _L)
            for l in range(_L):
                nxt = pair_loads(g * _L + l + 1) if l + 1 < _L else None
                plsc.store_scatter(tbuf, [colbase + l], pair_alu(*held))
                held = nxt
            terms = [plsc.load_gather(tbuf, [lane + k * _TP])
                     for k in range(_L)]
            while len(terms) > 1:
                terms = [terms[2 * t] + terms[2 * t + 1]
                         for t in range(len(terms) // 2)]
            dots = terms[0]
            sl = pl.ds(g * _L, _L)
            xsl = pl.ds(c * _CH + g * _L, _L)
            lnx = _ln16(xv[xsl])
            f = jnp.minimum(jnp.exp(_ALPHA * (lnx - _LN_XMAX)), 1.0)
            diff = dots + bib[slot, sl] + bjb[slot, sl] - lnx
            return acc + f * diff * diff

        with jax.named_scope(f"compute{c}"):
            acc = lax.fori_loop(0, _CH // _L, gbody, acc)
    accv[...] = acc
    pltpu.sync_copy(accv, out_hbm.at[wid])


def kernel(i, j, x, weight, weight_tilde, bias, bias_tilde):
    parts = _glove_sc(i, j, x, weight, weight_tilde, bias, bias_tilde)
    return jnp.sum(parts) / _B


# speedup vs baseline: 1.2520x; 1.0105x over previous
import functools

import jax
import jax.numpy as jnp
from jax import lax
from jax.experimental import pallas as pl
from jax.experimental.pallas import tpu as pltpu
from jax.experimental.pallas import tpu_sc as plsc

_V = 100000
_D = 128
_B = 16384
_XMAX = 100.0
_ALPHA = 0.75

_NC = 2
_NS = 16
_L = 16
_NW = _NC * _NS
_BPW = _B // _NW
_CH = 128
_NCH = _BPW // _CH
_TP = 17

_LN2 = 0.6931471805599453
_LN_XMAX = 4.605170185988092
_SQRT2 = 1.4142135623730951


_GDN = lax.GatherDimensionNumbers(
    offset_dims=(), collapsed_slice_dims=(0,), start_index_map=(0,))


def _permute(v, idx):
    return lax.gather(v, idx[:, None], _GDN, (1,),
                      mode=lax.GatherScatterMode.PROMISE_IN_BOUNDS)


def _ln16(v):
    bits = plsc.bitcast(v, jnp.int32)
    e = ((bits >> 23) & 0xFF) - 127
    m = plsc.bitcast((bits & 0x7FFFFF) | 0x3F800000, jnp.float32)
    big = m > _SQRT2
    m = jnp.where(big, m * 0.5, m)
    ef = (e + jnp.where(big, 1, 0)).astype(jnp.float32)
    s = (m - 1.0) / (m + 1.0)
    s2 = s * s
    p = 1.0 + s2 * ((1.0 / 3.0) + s2 * ((1.0 / 5.0) + s2 * (1.0 / 7.0)))
    return ef * _LN2 + 2.0 * s * p


@functools.partial(
    pl.kernel,
    out_type=jax.ShapeDtypeStruct((_NW, _L), jnp.float32),
    mesh=plsc.VectorSubcoreMesh(core_axis_name="c", subcore_axis_name="s"),
    compiler_params=pltpu.CompilerParams(needs_layout_passes=False),
    scratch_types=[
        pltpu.VMEM((_BPW,), jnp.int32),
        pltpu.VMEM((_BPW,), jnp.int32),
        pltpu.VMEM((_BPW,), jnp.float32),
        pltpu.VMEM((2, _CH, _D), jnp.float32),
        pltpu.VMEM((2, _CH, _D), jnp.float32),
        pltpu.VMEM((_BPW,), jnp.float32),
        pltpu.VMEM((_BPW,), jnp.float32),
        pltpu.VMEM((_L,), jnp.float32),
        pltpu.VMEM((_L * _TP,), jnp.float32),
        pltpu.VMEM((_L * _TP,), jnp.float32),
        pltpu.SemaphoreType.DMA,
        pltpu.SemaphoreType.DMA,
        pltpu.SemaphoreType.DMA,
    ],
)
def _glove_sc(i_hbm, j_hbm, x_hbm, w_hbm, wt_hbm, b_hbm, bt_hbm, out_hbm,
              iv, jv, xv, wib, wjb, bib, bjb, accv, tbuf, tbuf2,
              sem0, sem1, semb):
    wid = lax.axis_index("s") * _NC + lax.axis_index("c")
    base = wid * _BPW
    with jax.named_scope("stage_idx"):
        c0 = pltpu.async_copy(i_hbm.at[pl.ds(base, _BPW)], iv, sem0)
        c1 = pltpu.async_copy(j_hbm.at[pl.ds(base, _BPW)], jv, sem0)
        c2 = pltpu.async_copy(x_hbm.at[pl.ds(base, _BPW)], xv, sem0)
        c0.wait()
        c1.wait()
        c2.wait()

    sems = (sem0, sem1)

    def issue(c):
        slot = c % 2
        idx_i = iv.at[pl.ds(c * _CH, _CH)]
        idx_j = jv.at[pl.ds(c * _CH, _CH)]
        sem = sems[slot]
        return (
            pltpu.async_copy(w_hbm.at[idx_i], wib.at[slot], sem),
            pltpu.async_copy(wt_hbm.at[idx_j], wjb.at[slot], sem),
        )

    acc = jnp.zeros((_L,), jnp.float32)
    pending = {0: issue(0)}
    bias_cps = (pltpu.async_copy(b_hbm.at[iv], bib, semb),
                pltpu.async_copy(bt_hbm.at[jv], bjb, semb))
    for c in range(_NCH):
        slot = c % 2
        if c + 1 < _NCH:
            pending[c + 1] = issue(c + 1)
        with jax.named_scope(f"wait{c}"):
            for cp in pending.pop(c):
                cp.wait()
            if c == 0:
                for cp in bias_cps:
                    cp.wait()

        wiv = wib.at[slot]
        wjv = wjb.at[slot]

        def gbody(g, acc, slot=slot, wiv=wiv, wjv=wjv):
            lane = lax.iota(jnp.int32, _L)
            colbase = lane * _TP

            def pair_loads(p):
                return ([wiv[p, pl.ds(k * _L, _L)] for k in range(_D // _L)],
                        [wjv[p, pl.ds(k * _L, _L)] for k in range(_D // _L)])

            def pair_alu(wi, wj):
                m = [a * b for a, b in zip(wi, wj)]
                m = [m[2 * t] + m[2 * t + 1] for t in range(4)]
                m = [m[2 * t] + m[2 * t + 1] for t in range(2)]
                return m[0] + m[1]

            held = pair_loads(g * _L)
            for l in range(_L):
                nxt = pair_loads(g * _L + l + 1) if l + 1 < _L else None
                plsc.store_scatter(tbuf, [colbase + l], pair_alu(*held))
                held = nxt
            terms = [plsc.load_gather(tbuf, [lane + k * _TP])
                     for k in range(_L)]
            while len(terms) > 1:
                terms = [terms[2 * t] + terms[2 * t + 1]
                         for t in range(len(terms) // 2)]
            dots = terms[0]
            xsl = pl.ds(c * _CH + g * _L, _L)
            lnx = _ln16(xv[xsl])
            f = jnp.minimum(jnp.exp(_ALPHA * (lnx - _LN_XMAX)), 1.0)
            diff = dots + bib[xsl] + bjb[xsl] - lnx
            return acc + f * diff * diff

        with jax.named_scope(f"compute{c}"):
            acc = lax.fori_loop(0, _CH // _L, gbody, acc)
    accv[...] = acc
    pltpu.sync_copy(accv, out_hbm.at[wid])


def kernel(i, j, x, weight, weight_tilde, bias, bias_tilde):
    parts = _glove_sc(i, j, x, weight, weight_tilde, bias, bias_tilde)
    return jnp.sum(parts) / _B
